# Initial kernel scaffold; baseline (speedup 1.0000x reference)
import jax
import jax.numpy as jnp
from jax.experimental import pallas as pl


def kernel(gid, stop_mask, add_node_mask, add_edge_mask, table):
    raise NotImplementedError("write your pallas kernel here")


# speedup vs baseline: 1.8984x; 1.8984x over previous
import jax
import jax.numpy as jnp
from jax import lax
from jax.experimental import pallas as pl
from jax.experimental.pallas import tpu as pltpu, tpu_sc as plsc

N_STATES = 200000
B = 16384
N_NODE = 9
N_TYPE = 8
N_EDGE = 36
PER = 1 + N_NODE * N_TYPE + N_EDGE + 1
NODE_W = N_NODE * N_TYPE
NC, NS = 2, 16
NW = NC * NS
NB = B // NW
CH = 256
RW = 128
TROWS = N_STATES * PER // RW
GCH = 128

_mesh = plsc.VectorSubcoreMesh(core_axis_name="c", subcore_axis_name="s")


def _masked(v, m):
    return v * m + 1000.0 * (m - 1.0)


@pl.kernel(
    out_type=(
        jax.ShapeDtypeStruct((B,), jnp.float32),
        jax.ShapeDtypeStruct((B * NODE_W,), jnp.float32),
        jax.ShapeDtypeStruct((B * N_EDGE,), jnp.float32),
        jax.ShapeDtypeStruct((B,), jnp.float32),
    ),
    mesh=_mesh,
    compiler_params=pltpu.CompilerParams(needs_layout_passes=False),
    scratch_types=[
        pltpu.VMEM((2 * CH, RW), jnp.float32),
        pltpu.VMEM((CH,), jnp.int32),
        pltpu.VMEM((2 * CH,), jnp.int32),
        pltpu.VMEM((CH * NODE_W,), jnp.float32),
        pltpu.VMEM((CH * N_EDGE,), jnp.float32),
        pltpu.VMEM((CH,), jnp.float32),
        pltpu.VMEM((CH,), jnp.float32),
        pltpu.SemaphoreType.DMA,
    ],
)
def _sc_lookup(table_ref, gid_ref, sm_ref, nm_ref, em_ref,
               stop_out, node_out, edge_out, logf_out,
               rows_v, gid_v, idx_v, nm_v, em_v, sm_v, logf_v, sem):
    wid = lax.axis_index("s") * NC + lax.axis_index("c")
    iota = lax.iota(jnp.int32, 16)

    for ch in range(NB // CH):
        rb = wid * NB + ch * CH

        pltpu.sync_copy(gid_ref.at[pl.ds(rb, CH)], gid_v)

        def idx_body(k, _):
            rvec = (16 * k + iota) >> 1
            gv = plsc.load_gather(gid_v, [rvec])
            a = ((gv * PER) >> 7) + (iota & 1)
            idx_v[pl.ds(16 * k, 16)] = jnp.minimum(a, TROWS - 1)
            return _
        lax.fori_loop(0, 2 * CH // 16, idx_body, 0)

        gathers = [
            pltpu.async_copy(table_ref.at[idx_v.at[pl.ds(k * GCH, GCH)]],
                             rows_v.at[pl.ds(k * GCH, GCH)], sem)
            for k in range(2 * CH // GCH)
        ]
        pltpu.sync_copy(sm_ref.at[pl.ds(rb, CH)], sm_v)
        pltpu.sync_copy(nm_ref.at[pl.ds(rb * NODE_W, CH * NODE_W)], nm_v)
        pltpu.sync_copy(em_ref.at[pl.ds(rb * N_EDGE, CH * N_EDGE)], em_v)
        for g in gathers:
            g.wait()

        def stop_body(k, _):
            rvec = 16 * k + iota
            gv = plsc.load_gather(gid_v, [rvec])
            f = 2 * RW * rvec + ((gv * PER) & (RW - 1))
            v = plsc.load_gather(rows_v, [f >> 7, f & (RW - 1)])
            m = sm_v[pl.ds(16 * k, 16)]
            sm_v[pl.ds(16 * k, 16)] = _masked(v, m)
            f2 = f + (PER - 1)
            logf_v[pl.ds(16 * k, 16)] = plsc.load_gather(
                rows_v, [f2 >> 7, f2 & (RW - 1)])
            return _
        lax.fori_loop(0, CH // 16, stop_body, 0)

        def node_body(p, _):
            sa = (plsc.load_gather(gid_v, [iota * 0 + 2 * p]) * PER) & (RW - 1)
            sb = (plsc.load_gather(gid_v, [iota * 0 + 2 * p + 1]) * PER) & (RW - 1)
            for j in range(9):
                e = 16 * j + iota
                half = (e >= NODE_W).astype(jnp.int32)
                f = (2 * RW * (2 * p) + 2 * RW * half
                     + jnp.where(e >= NODE_W, sb, sa)
                     + 1 + e - NODE_W * half)
                v = plsc.load_gather(rows_v, [f >> 7, f & (RW - 1)])
                off = 144 * p + 16 * j
                m = nm_v[pl.ds(off, 16)]
                nm_v[pl.ds(off, 16)] = _masked(v, m)
            return _
        lax.fori_loop(0, CH // 2, node_body, 0)

        def edge_body(p, _):
            s = [(plsc.load_gather(gid_v, [iota * 0 + 4 * p + q]) * PER)
                 & (RW - 1) for q in range(4)]
            for j in range(9):
                e = 16 * j + iota
                r4 = ((e >= N_EDGE).astype(jnp.int32)
                      + (e >= 2 * N_EDGE).astype(jnp.int32)
                      + (e >= 3 * N_EDGE).astype(jnp.int32))
                s_sel = jnp.where(
                    e >= 3 * N_EDGE, s[3],
                    jnp.where(e >= 2 * N_EDGE, s[2],
                              jnp.where(e >= N_EDGE, s[1], s[0])))
                f = (2 * RW * (4 * p) + 2 * RW * r4 + s_sel
                     + (1 + NODE_W) + e - N_EDGE * r4)
                v = plsc.load_gather(rows_v, [f >> 7, f & (RW - 1)])
                off = 144 * p + 16 * j
                m = em_v[pl.ds(off, 16)]
                em_v[pl.ds(off, 16)] = _masked(v, m)
            return _
        lax.fori_loop(0, CH // 4, edge_body, 0)

        pltpu.sync_copy(sm_v, stop_out.at[pl.ds(rb, CH)])
        pltpu.sync_copy(nm_v, node_out.at[pl.ds(rb * NODE_W, CH * NODE_W)])
        pltpu.sync_copy(em_v, edge_out.at[pl.ds(rb * N_EDGE, CH * N_EDGE)])
        pltpu.sync_copy(logf_v, logf_out.at[pl.ds(rb, CH)])


def kernel(gid, stop_mask, add_node_mask, add_edge_mask, table):
    t128 = table.reshape(TROWS, RW)
    gid1 = gid.astype(jnp.int32)
    smf = stop_mask.astype(jnp.float32).reshape(-1)
    nmf = add_node_mask.astype(jnp.float32).reshape(-1)
    emf = add_edge_mask.astype(jnp.float32).reshape(-1)
    stop_f, node_f, edge_f, logf_f = _sc_lookup(t128, gid1, smf, nmf, emf)
    return (stop_f.reshape(B, 1),
            node_f.reshape(B * N_NODE, N_TYPE),
            edge_f.reshape(B * N_EDGE, 1),
            logf_f.reshape(B, 1))


# speedup vs baseline: 1.9067x; 1.0044x over previous
import jax
import jax.numpy as jnp
from jax import lax
from jax.experimental import pallas as pl
from jax.experimental.pallas import tpu as pltpu, tpu_sc as plsc

N_STATES = 200000
B = 16384
N_NODE = 9
N_TYPE = 8
N_EDGE = 36
PER = 1 + N_NODE * N_TYPE + N_EDGE + 1
NODE_W = N_NODE * N_TYPE
NC, NS = 2, 16
NW = NC * NS
NB = B // NW
CH = 256
RW = 128
TROWS = N_STATES * PER // RW
GCH = 128

_mesh = plsc.VectorSubcoreMesh(core_axis_name="c", subcore_axis_name="s")


def _masked(v, m):
    return v * m + 1000.0 * (m - 1.0)


@pl.kernel(
    out_type=(
        jax.ShapeDtypeStruct((B,), jnp.float32),
        jax.ShapeDtypeStruct((B * NODE_W,), jnp.float32),
        jax.ShapeDtypeStruct((B * N_EDGE,), jnp.float32),
        jax.ShapeDtypeStruct((B,), jnp.float32),
    ),
    mesh=_mesh,
    compiler_params=pltpu.CompilerParams(needs_layout_passes=False),
    scratch_types=[
        pltpu.VMEM((2 * CH, RW), jnp.float32),
        pltpu.VMEM((CH,), jnp.int32),
        pltpu.VMEM((2 * CH,), jnp.int32),
        pltpu.VMEM((CH * NODE_W,), jnp.int32),
        pltpu.VMEM((CH * N_EDGE,), jnp.int32),
        pltpu.VMEM((CH,), jnp.int32),
        pltpu.VMEM((CH * NODE_W,), jnp.float32),
        pltpu.VMEM((CH * N_EDGE,), jnp.float32),
        pltpu.VMEM((CH,), jnp.float32),
        pltpu.VMEM((CH,), jnp.float32),
        pltpu.SemaphoreType.DMA,
    ],
)
def _sc_lookup(table_ref, gid_ref, sm_ref, nm_ref, em_ref,
               stop_out, node_out, edge_out, logf_out,
               rows_v, gid_v, idx_v, nm_v, em_v, sm_v,
               nmo_v, emo_v, smo_v, logf_v, sem):
    wid = lax.axis_index("s") * NC + lax.axis_index("c")
    iota = lax.iota(jnp.int32, 16)

    for ch in range(NB // CH):
        rb = wid * NB + ch * CH

        pltpu.sync_copy(gid_ref.at[pl.ds(rb, CH)], gid_v)

        def idx_body(k, _):
            rvec = (16 * k + iota) >> 1
            gv = plsc.load_gather(gid_v, [rvec])
            a = ((gv * PER) >> 7) + (iota & 1)
            idx_v[pl.ds(16 * k, 16)] = jnp.minimum(a, TROWS - 1)
            return _
        lax.fori_loop(0, 2 * CH // 16, idx_body, 0)

        gathers = [
            pltpu.async_copy(table_ref.at[idx_v.at[pl.ds(k * GCH, GCH)]],
                             rows_v.at[pl.ds(k * GCH, GCH)], sem)
            for k in range(2 * CH // GCH)
        ]
        pltpu.sync_copy(sm_ref.at[pl.ds(rb, CH)], sm_v)
        pltpu.sync_copy(nm_ref.at[pl.ds(rb * NODE_W, CH * NODE_W)], nm_v)
        pltpu.sync_copy(em_ref.at[pl.ds(rb * N_EDGE, CH * N_EDGE)], em_v)
        for g in gathers:
            g.wait()

        def stop_body(k, _):
            rvec = 16 * k + iota
            gv = plsc.load_gather(gid_v, [rvec])
            f = 2 * RW * rvec + ((gv * PER) & (RW - 1))
            v = plsc.load_gather(rows_v, [f >> 7, f & (RW - 1)])
            m = sm_v[pl.ds(16 * k, 16)].astype(jnp.float32)
            smo_v[pl.ds(16 * k, 16)] = _masked(v, m)
            f2 = f + (PER - 1)
            logf_v[pl.ds(16 * k, 16)] = plsc.load_gather(
                rows_v, [f2 >> 7, f2 & (RW - 1)])
            return _
        lax.fori_loop(0, CH // 16, stop_body, 0)

        def node_body(p, _):
            sa = (plsc.load_gather(gid_v, [iota * 0 + 2 * p]) * PER) & (RW - 1)
            sb = (plsc.load_gather(gid_v, [iota * 0 + 2 * p + 1]) * PER) & (RW - 1)
            for j in range(9):
                e = 16 * j + iota
                half = (e >= NODE_W).astype(jnp.int32)
                f = (2 * RW * (2 * p) + 2 * RW * half
                     + jnp.where(e >= NODE_W, sb, sa)
                     + 1 + e - NODE_W * half)
                v = plsc.load_gather(rows_v, [f >> 7, f & (RW - 1)])
                off = 144 * p + 16 * j
                m = nm_v[pl.ds(off, 16)].astype(jnp.float32)
                nmo_v[pl.ds(off, 16)] = _masked(v, m)
            return _
        lax.fori_loop(0, CH // 2, node_body, 0)

        def edge_body(p, _):
            s = [(plsc.load_gather(gid_v, [iota * 0 + 4 * p + q]) * PER)
                 & (RW - 1) for q in range(4)]
            for j in range(9):
                e = 16 * j + iota
                r4 = ((e >= N_EDGE).astype(jnp.int32)
                      + (e >= 2 * N_EDGE).astype(jnp.int32)
                      + (e >= 3 * N_EDGE).astype(jnp.int32))
                s_sel = jnp.where(
                    e >= 3 * N_EDGE, s[3],
                    jnp.where(e >= 2 * N_EDGE, s[2],
                              jnp.where(e >= N_EDGE, s[1], s[0])))
                f = (2 * RW * (4 * p) + 2 * RW * r4 + s_sel
                     + (1 + NODE_W) + e - N_EDGE * r4)
                v = plsc.load_gather(rows_v, [f >> 7, f & (RW - 1)])
                off = 144 * p + 16 * j
                m = em_v[pl.ds(off, 16)].astype(jnp.float32)
                emo_v[pl.ds(off, 16)] = _masked(v, m)
            return _
        lax.fori_loop(0, CH // 4, edge_body, 0)

        pltpu.sync_copy(smo_v, stop_out.at[pl.ds(rb, CH)])
        pltpu.sync_copy(nmo_v, node_out.at[pl.ds(rb * NODE_W, CH * NODE_W)])
        pltpu.sync_copy(emo_v, edge_out.at[pl.ds(rb * N_EDGE, CH * N_EDGE)])
        pltpu.sync_copy(logf_v, logf_out.at[pl.ds(rb, CH)])


def kernel(gid, stop_mask, add_node_mask, add_edge_mask, table):
    t128 = table.reshape(TROWS, RW)
    gid1 = gid.astype(jnp.int32)
    smf = stop_mask.astype(jnp.int32).reshape(-1)
    nmf = add_node_mask.astype(jnp.int32).reshape(-1)
    emf = add_edge_mask.astype(jnp.int32).reshape(-1)
    stop_f, node_f, edge_f, logf_f = _sc_lookup(t128, gid1, smf, nmf, emf)
    return (stop_f.reshape(B, 1),
            node_f.reshape(B * N_NODE, N_TYPE),
            edge_f.reshape(B * N_EDGE, 1),
            logf_f.reshape(B, 1))
